# baseline (speedup 1.0000x reference)
import jax
import jax.numpy as jnp
from jax.experimental import pallas as pl
from jax.experimental.pallas import tpu as pltpu

N = 2048
F_IN = 90
HEADS = 4
HID1 = 64
HID2 = 128
OUT_DIM = 10
NBLK = 2
BLK = N // NBLK
LOG2E = 1.4426950408889634


def _leaky(x, slope):
    return jnp.where(x >= 0, x, slope * x)


def _gcn_kernel(idx_ref, adj_ref, x_ref, wg_ref, mcol_ref, mrow_ref, w2_ref,
                bg_ref, b2_ref, gate_ref, out_ref,
                wxp_scr, ac_scr, u_scr, den_scr, agg_scr, deg_scr,
                b_scr):
    del idx_ref
    i = pl.program_id(1)

    @pl.when(i == 0)
    def _init_graph():
        x = x_ref[0]
        wx = jax.lax.dot_general(x, wg_ref[...], (((1,), (0,)), ((), ())),
                                 preferred_element_type=jnp.float32)
        ac = jax.lax.dot_general(wx, mcol_ref[...], (((1,), (0,)), ((), ())),
                                 preferred_element_type=jnp.float32)
        ac_scr[...] = jnp.concatenate(
            [ac[:, 0:HEADS] * LOG2E, ac[:, 0:HEADS] * (0.2 * LOG2E)], axis=1)
        ar = jax.lax.dot_general(mrow_ref[...], wx, (((1,), (1,)), ((), ())),
                                 preferred_element_type=jnp.float32)
        for h in range(HEADS):
            ad_r = ar[HEADS + h:HEADS + h + 1, :]
            gmax = jnp.max(ac[:, h:h + 1])
            u = _leaky(gmax + ad_r, 0.2)
            u_scr[h:h + 1, :] = (ad_r - u) * LOG2E
            u_scr[HEADS + h:HEADS + h + 1, :] = (ad_r * 0.2 - u) * LOG2E
        wxp_scr[...] = wx.astype(jnp.bfloat16)
        den_scr[...] = jnp.zeros(den_scr.shape, jnp.float32)
        agg_scr[...] = jnp.zeros(agg_scr.shape, jnp.float32)

    @pl.when(i < NBLK)
    def _attention_block():
        row0 = i * BLK
        a_blk = adj_ref[0]
        rid = row0 + jax.lax.broadcasted_iota(jnp.int32, (BLK, N), 0)
        cid = jax.lax.broadcasted_iota(jnp.int32, (BLK, N), 1)
        b_blk = jnp.where(jnp.logical_or(a_blk > 0.5, rid == cid),
                          jnp.float32(1.0), jnp.float32(0.0))
        b_scr[pl.ds(row0, BLK), :] = b_blk.astype(jnp.bfloat16)
        ones_col = jnp.full((N, 1), 1.0, jnp.float32)
        deg_scr[pl.ds(row0, BLK), :] = jax.lax.dot_general(
            b_blk, ones_col, (((1,), (0,)), ((), ())),
            preferred_element_type=jnp.float32)
        ones_row = jnp.full((1, BLK), 1.0, jnp.bfloat16)
        for h in range(HEADS):
            t1 = ac_scr[pl.ds(row0, BLK), h:h + 1] + u_scr[h:h + 1, :]
            t2 = (ac_scr[pl.ds(row0, BLK), HEADS + h:HEADS + h + 1]
                  + u_scr[HEADS + h:HEADS + h + 1, :])
            p = (b_blk * jnp.exp2(jnp.maximum(t1, t2))).astype(jnp.bfloat16)
            den_scr[h:h + 1, :] = den_scr[h:h + 1, :] + jax.lax.dot_general(
                ones_row, p, (((1,), (0,)), ((), ())),
                preferred_element_type=jnp.float32)
            wxh = wxp_scr[pl.ds(row0, BLK), pl.ds(h * HID1, HID1)]
            agg_scr[h] = agg_scr[h] + jax.lax.dot_general(
                p, wxh, (((0,), (0,)), ((), ())),
                preferred_element_type=jnp.float32)

    @pl.when(i == NBLK)
    def _finalize():
        inv = jnp.float32(1.0) / (den_scr[...] + jnp.float32(1e-16))
        eye8 = jnp.where(
            jax.lax.broadcasted_iota(jnp.int32, (8, 8), 0)
            == jax.lax.broadcasted_iota(jnp.int32, (8, 8), 1),
            jnp.float32(1.0), jnp.float32(0.0))
        inv_col = jax.lax.dot_general(inv, eye8, (((0,), (0,)), ((), ())),
                                      preferred_element_type=jnp.float32)
        x1 = agg_scr[0] * inv_col[:, 0:1]
        for h in range(1, HEADS):
            x1 = x1 + agg_scr[h] * inv_col[:, h:h + 1]
        x1 = x1 * jnp.float32(1.0 / HEADS) + bg_ref[...]
        x1 = _leaky(x1, 0.2)
        h2 = jax.lax.dot_general(x1, w2_ref[...], (((1,), (0,)), ((), ())),
                                 preferred_element_type=jnp.float32)
        dinv = jnp.float32(1.0) / jnp.sqrt(deg_scr[...])
        h2s = (h2 * dinv).astype(jnp.bfloat16)
        y = jax.lax.dot_general(b_scr[...], h2s, (((1,), (0,)), ((), ())),
                                preferred_element_type=jnp.float32)
        x2 = dinv * y + b2_ref[...]
        out_ref[0] = _leaky(x2, 0.2) * gate_ref[0]


def _cls_kernel(g_ref, wt_ref, bc_ref, out_ref):
    logits = jax.lax.dot_general(g_ref[...], wt_ref[...],
                                 (((1,), (1,)), ((), ())),
                                 preferred_element_type=jnp.float32)
    logits = logits + bc_ref[...]
    mx = jnp.max(logits, axis=1, keepdims=True)
    sh = logits - mx
    out_ref[...] = sh - jnp.log(jnp.sum(jnp.exp(sh), axis=1, keepdims=True))


@jax.jit
def kernel(features0, features1, adjs, action, index, W_gat, a_src, a_dst,
           b_gat, W_fc2, b_fc2, W_cls, b_cls):
    del features1
    idx = index.astype(jnp.int32)
    ngr = idx.shape[0]

    eye4 = jnp.eye(HEADS, dtype=jnp.float32)
    m_src = jnp.einsum('hk,hg->hkg', a_src, eye4).reshape(HEADS * HID1, HEADS)
    m_dst = jnp.einsum('hk,hg->hkg', a_dst, eye4).reshape(HEADS * HID1, HEADS)
    m_col = jnp.concatenate([m_src, m_dst], axis=1)
    m_row = m_col.T
    gate = jnp.where(action == 1, jnp.float32(1.0),
                     jnp.float32(0.0)).reshape(1,)

    grid = (ngr, NBLK + 1)
    x2_all = pl.pallas_call(
        _gcn_kernel,
        grid_spec=pltpu.PrefetchScalarGridSpec(
            num_scalar_prefetch=1,
            grid=grid,
            in_specs=[
                pl.BlockSpec(
                    (1, BLK, N),
                    lambda g, i, idx_ref: (idx_ref[g],
                                           jnp.minimum(i, NBLK - 1), 0)),
                pl.BlockSpec((1, N, F_IN),
                             lambda g, i, idx_ref: (idx_ref[g], 0, 0)),
                pl.BlockSpec((F_IN, HEADS * HID1),
                             lambda g, i, idx_ref: (0, 0)),
                pl.BlockSpec((HEADS * HID1, 2 * HEADS),
                             lambda g, i, idx_ref: (0, 0)),
                pl.BlockSpec((2 * HEADS, HEADS * HID1),
                             lambda g, i, idx_ref: (0, 0)),
                pl.BlockSpec((HID1, HID2), lambda g, i, idx_ref: (0, 0)),
                pl.BlockSpec((1, HID1), lambda g, i, idx_ref: (0, 0)),
                pl.BlockSpec((1, HID2), lambda g, i, idx_ref: (0, 0)),
                pl.BlockSpec(memory_space=pltpu.SMEM),
            ],
            out_specs=pl.BlockSpec((1, N, HID2),
                                   lambda g, i, idx_ref: (g, 0, 0)),
            scratch_shapes=[
                pltpu.VMEM((N, HEADS * HID1), jnp.bfloat16),
                pltpu.VMEM((N, 2 * HEADS), jnp.float32),
                pltpu.VMEM((2 * HEADS, N), jnp.float32),
                pltpu.VMEM((2 * HEADS, N), jnp.float32),
                pltpu.VMEM((HEADS, N, HID1), jnp.float32),
                pltpu.VMEM((N, 1), jnp.float32),
                pltpu.VMEM((N, N), jnp.bfloat16),
            ],
        ),
        out_shape=jax.ShapeDtypeStruct((ngr, N, HID2), jnp.float32),
        compiler_params=pltpu.CompilerParams(
            dimension_semantics=("parallel", "arbitrary")),
    )(idx, adjs, features0, W_gat, m_col, m_row, W_fc2,
      b_gat.reshape(1, HID1), b_fc2.reshape(1, HID2), gate)

    g_mat = x2_all.reshape(ngr, N * HID2)
    logp = pl.pallas_call(
        _cls_kernel,
        out_shape=jax.ShapeDtypeStruct((ngr, OUT_DIM), jnp.float32),
    )(g_mat, W_cls.T, b_cls.reshape(1, OUT_DIM))

    return (x2_all[ngr - 1], logp)


# speedup vs baseline: 1.0125x; 1.0125x over previous
import jax
import jax.numpy as jnp
from jax.experimental import pallas as pl
from jax.experimental.pallas import tpu as pltpu

N = 2048
F_IN = 90
HEADS = 4
HID1 = 64
HID2 = 128
OUT_DIM = 10
NBLK = 4
BLK = N // NBLK
LOG2E = 1.4426950408889634


def _leaky(x, slope):
    return jnp.where(x >= 0, x, slope * x)


def _gcn_kernel(idx_ref, adj_ref, x_ref, wg_ref, mcol_ref, mrow_ref, w2_ref,
                bg_ref, b2_ref, gate_ref, out_ref,
                wxp_scr, ac_scr, u_scr, den_scr, agg_scr, deg_scr,
                b_scr):
    del idx_ref
    i = pl.program_id(1)

    @pl.when(i == 0)
    def _init_graph():
        x = x_ref[0]
        wx = jax.lax.dot_general(x, wg_ref[...], (((1,), (0,)), ((), ())),
                                 preferred_element_type=jnp.float32)
        ac = jax.lax.dot_general(wx, mcol_ref[...], (((1,), (0,)), ((), ())),
                                 preferred_element_type=jnp.float32)
        ac_scr[...] = jnp.concatenate(
            [ac[:, 0:HEADS] * LOG2E, ac[:, 0:HEADS] * (0.2 * LOG2E)], axis=1)
        ar = jax.lax.dot_general(mrow_ref[...], wx, (((1,), (1,)), ((), ())),
                                 preferred_element_type=jnp.float32)
        for h in range(HEADS):
            ad_r = ar[HEADS + h:HEADS + h + 1, :]
            gmax = jnp.max(ac[:, h:h + 1])
            u = _leaky(gmax + ad_r, 0.2)
            u_scr[h:h + 1, :] = (ad_r - u) * LOG2E
            u_scr[HEADS + h:HEADS + h + 1, :] = (ad_r * 0.2 - u) * LOG2E
        wxp_scr[...] = wx.astype(jnp.bfloat16)
        den_scr[...] = jnp.zeros(den_scr.shape, jnp.float32)
        agg_scr[...] = jnp.zeros(agg_scr.shape, jnp.float32)

    @pl.when(i < NBLK)
    def _attention_block():
        row0 = i * BLK
        a_blk = adj_ref[0]
        rid = row0 + jax.lax.broadcasted_iota(jnp.int32, (BLK, N), 0)
        cid = jax.lax.broadcasted_iota(jnp.int32, (BLK, N), 1)
        b_blk = jnp.where(jnp.logical_or(a_blk > 0.5, rid == cid),
                          jnp.float32(1.0), jnp.float32(0.0))
        b_scr[pl.ds(row0, BLK), :] = b_blk.astype(jnp.bfloat16)
        ones_col = jnp.full((N, 1), 1.0, jnp.float32)
        deg_scr[pl.ds(row0, BLK), :] = jax.lax.dot_general(
            b_blk, ones_col, (((1,), (0,)), ((), ())),
            preferred_element_type=jnp.float32)
        ones_row = jnp.full((1, BLK), 1.0, jnp.bfloat16)
        for h in range(HEADS):
            t1 = ac_scr[pl.ds(row0, BLK), h:h + 1] + u_scr[h:h + 1, :]
            t2 = (ac_scr[pl.ds(row0, BLK), HEADS + h:HEADS + h + 1]
                  + u_scr[HEADS + h:HEADS + h + 1, :])
            p = (b_blk * jnp.exp2(jnp.maximum(t1, t2))).astype(jnp.bfloat16)
            den_scr[h:h + 1, :] = den_scr[h:h + 1, :] + jax.lax.dot_general(
                ones_row, p, (((1,), (0,)), ((), ())),
                preferred_element_type=jnp.float32)
            wxh = wxp_scr[pl.ds(row0, BLK), pl.ds(h * HID1, HID1)]
            agg_scr[h] = agg_scr[h] + jax.lax.dot_general(
                p, wxh, (((0,), (0,)), ((), ())),
                preferred_element_type=jnp.float32)

    @pl.when(i == NBLK)
    def _finalize():
        inv = jnp.float32(1.0) / (den_scr[...] + jnp.float32(1e-16))
        eye8 = jnp.where(
            jax.lax.broadcasted_iota(jnp.int32, (8, 8), 0)
            == jax.lax.broadcasted_iota(jnp.int32, (8, 8), 1),
            jnp.float32(1.0), jnp.float32(0.0))
        inv_col = jax.lax.dot_general(inv, eye8, (((0,), (0,)), ((), ())),
                                      preferred_element_type=jnp.float32)
        x1 = agg_scr[0] * inv_col[:, 0:1]
        for h in range(1, HEADS):
            x1 = x1 + agg_scr[h] * inv_col[:, h:h + 1]
        x1 = x1 * jnp.float32(1.0 / HEADS) + bg_ref[...]
        x1 = _leaky(x1, 0.2)
        h2 = jax.lax.dot_general(x1, w2_ref[...], (((1,), (0,)), ((), ())),
                                 preferred_element_type=jnp.float32)
        dinv = jnp.float32(1.0) / jnp.sqrt(deg_scr[...])
        h2s = (h2 * dinv).astype(jnp.bfloat16)
        y = jax.lax.dot_general(b_scr[...], h2s, (((1,), (0,)), ((), ())),
                                preferred_element_type=jnp.float32)
        x2 = dinv * y + b2_ref[...]
        out_ref[0] = _leaky(x2, 0.2) * gate_ref[0]


def _cls_kernel(g_ref, wt_ref, bc_ref, out_ref):
    logits = jax.lax.dot_general(g_ref[...], wt_ref[...],
                                 (((1,), (1,)), ((), ())),
                                 preferred_element_type=jnp.float32)
    logits = logits + bc_ref[...]
    mx = jnp.max(logits, axis=1, keepdims=True)
    sh = logits - mx
    out_ref[...] = sh - jnp.log(jnp.sum(jnp.exp(sh), axis=1, keepdims=True))


@jax.jit
def kernel(features0, features1, adjs, action, index, W_gat, a_src, a_dst,
           b_gat, W_fc2, b_fc2, W_cls, b_cls):
    del features1
    idx = index.astype(jnp.int32)
    ngr = idx.shape[0]

    eye4 = jnp.eye(HEADS, dtype=jnp.float32)
    m_src = jnp.einsum('hk,hg->hkg', a_src, eye4).reshape(HEADS * HID1, HEADS)
    m_dst = jnp.einsum('hk,hg->hkg', a_dst, eye4).reshape(HEADS * HID1, HEADS)
    m_col = jnp.concatenate([m_src, m_dst], axis=1)
    m_row = m_col.T
    gate = jnp.where(action == 1, jnp.float32(1.0),
                     jnp.float32(0.0)).reshape(1,)

    grid = (ngr, NBLK + 1)
    x2_all = pl.pallas_call(
        _gcn_kernel,
        grid_spec=pltpu.PrefetchScalarGridSpec(
            num_scalar_prefetch=1,
            grid=grid,
            in_specs=[
                pl.BlockSpec(
                    (1, BLK, N),
                    lambda g, i, idx_ref: (idx_ref[g],
                                           jnp.minimum(i, NBLK - 1), 0)),
                pl.BlockSpec((1, N, F_IN),
                             lambda g, i, idx_ref: (idx_ref[g], 0, 0)),
                pl.BlockSpec((F_IN, HEADS * HID1),
                             lambda g, i, idx_ref: (0, 0)),
                pl.BlockSpec((HEADS * HID1, 2 * HEADS),
                             lambda g, i, idx_ref: (0, 0)),
                pl.BlockSpec((2 * HEADS, HEADS * HID1),
                             lambda g, i, idx_ref: (0, 0)),
                pl.BlockSpec((HID1, HID2), lambda g, i, idx_ref: (0, 0)),
                pl.BlockSpec((1, HID1), lambda g, i, idx_ref: (0, 0)),
                pl.BlockSpec((1, HID2), lambda g, i, idx_ref: (0, 0)),
                pl.BlockSpec(memory_space=pltpu.SMEM),
            ],
            out_specs=pl.BlockSpec((1, N, HID2),
                                   lambda g, i, idx_ref: (g, 0, 0)),
            scratch_shapes=[
                pltpu.VMEM((N, HEADS * HID1), jnp.bfloat16),
                pltpu.VMEM((N, 2 * HEADS), jnp.float32),
                pltpu.VMEM((2 * HEADS, N), jnp.float32),
                pltpu.VMEM((2 * HEADS, N), jnp.float32),
                pltpu.VMEM((HEADS, N, HID1), jnp.float32),
                pltpu.VMEM((N, 1), jnp.float32),
                pltpu.VMEM((N, N), jnp.bfloat16),
            ],
        ),
        out_shape=jax.ShapeDtypeStruct((ngr, N, HID2), jnp.float32),
        compiler_params=pltpu.CompilerParams(
            dimension_semantics=("parallel", "arbitrary")),
    )(idx, adjs, features0, W_gat, m_col, m_row, W_fc2,
      b_gat.reshape(1, HID1), b_fc2.reshape(1, HID2), gate)

    g_mat = x2_all.reshape(ngr, N * HID2)
    logp = pl.pallas_call(
        _cls_kernel,
        out_shape=jax.ShapeDtypeStruct((ngr, OUT_DIM), jnp.float32),
    )(g_mat, W_cls.T, b_cls.reshape(1, OUT_DIM))

    return (x2_all[ngr - 1], logp)


# speedup vs baseline: 1.0608x; 1.0477x over previous
import jax
import jax.numpy as jnp
from jax.experimental import pallas as pl
from jax.experimental.pallas import tpu as pltpu

N = 2048
F_IN = 90
HEADS = 4
HID1 = 64
HID2 = 128
OUT_DIM = 10
NBLK = 4
BLK = N // NBLK
LOG2E = 1.4426950408889634


def _leaky(x, slope):
    return jnp.where(x >= 0, x, slope * x)


def _gcn_kernel(idx_ref, adj_ref, x_ref, wg_ref, mcol_ref, mrow_ref, w2_ref,
                bg_ref, b2_ref, gate_ref, out_ref,
                wxp_scr, ac_scr, u_scr, den_scr, agg_scr, deg_scr,
                b_scr):
    del idx_ref
    i = pl.program_id(1)

    @pl.when(i == 0)
    def _init_graph():
        x = x_ref[0]
        wx = jax.lax.dot_general(x, wg_ref[...], (((1,), (0,)), ((), ())),
                                 preferred_element_type=jnp.float32)
        ac = jax.lax.dot_general(wx, mcol_ref[...], (((1,), (0,)), ((), ())),
                                 preferred_element_type=jnp.float32)
        ac_scr[...] = jnp.concatenate(
            [ac[:, 0:HEADS] * LOG2E, ac[:, 0:HEADS] * (0.2 * LOG2E)],
            axis=1).astype(jnp.bfloat16)
        ar = jax.lax.dot_general(mrow_ref[...], wx, (((1,), (1,)), ((), ())),
                                 preferred_element_type=jnp.float32)
        for h in range(HEADS):
            ad_r = ar[HEADS + h:HEADS + h + 1, :]
            gmax = jnp.max(ac[:, h:h + 1])
            u = _leaky(gmax + ad_r, 0.2)
            u_scr[h:h + 1, :] = ((ad_r - u) * LOG2E).astype(jnp.bfloat16)
            u_scr[HEADS + h:HEADS + h + 1, :] = (
                (ad_r * 0.2 - u) * LOG2E).astype(jnp.bfloat16)
        wxp_scr[...] = wx.astype(jnp.bfloat16)
        den_scr[...] = jnp.zeros(den_scr.shape, jnp.float32)
        agg_scr[...] = jnp.zeros(agg_scr.shape, jnp.float32)

    @pl.when(i < NBLK)
    def _attention_block():
        row0 = i * BLK
        a_blk = adj_ref[0]
        rid = row0 + jax.lax.broadcasted_iota(jnp.int32, (BLK, N), 0)
        cid = jax.lax.broadcasted_iota(jnp.int32, (BLK, N), 1)
        b_blk = jnp.where(jnp.logical_or(a_blk > 0.5, rid == cid),
                          jnp.float32(1.0), jnp.float32(0.0))
        b_scr[pl.ds(row0, BLK), :] = b_blk.astype(jnp.bfloat16)
        ones_col = jnp.full((N, 1), 1.0, jnp.float32)
        deg_scr[pl.ds(row0, BLK), :] = jax.lax.dot_general(
            b_blk, ones_col, (((1,), (0,)), ((), ())),
            preferred_element_type=jnp.float32)
        ones_row = jnp.full((1, BLK), 1.0, jnp.bfloat16)
        b16 = b_blk.astype(jnp.bfloat16)
        for h in range(HEADS):
            t1 = ac_scr[pl.ds(row0, BLK), h:h + 1] + u_scr[h:h + 1, :]
            t2 = (ac_scr[pl.ds(row0, BLK), HEADS + h:HEADS + h + 1]
                  + u_scr[HEADS + h:HEADS + h + 1, :])
            p = b16 * jnp.exp2(jnp.maximum(t1, t2))
            den_scr[h:h + 1, :] = den_scr[h:h + 1, :] + jax.lax.dot_general(
                ones_row, p, (((1,), (0,)), ((), ())),
                preferred_element_type=jnp.float32)
            wxh = wxp_scr[pl.ds(row0, BLK), pl.ds(h * HID1, HID1)]
            agg_scr[h] = agg_scr[h] + jax.lax.dot_general(
                p, wxh, (((0,), (0,)), ((), ())),
                preferred_element_type=jnp.float32)

    @pl.when(i == NBLK)
    def _finalize():
        inv = jnp.float32(1.0) / (den_scr[...] + jnp.float32(1e-16))
        eye8 = jnp.where(
            jax.lax.broadcasted_iota(jnp.int32, (8, 8), 0)
            == jax.lax.broadcasted_iota(jnp.int32, (8, 8), 1),
            jnp.float32(1.0), jnp.float32(0.0))
        inv_col = jax.lax.dot_general(inv, eye8, (((0,), (0,)), ((), ())),
                                      preferred_element_type=jnp.float32)
        x1 = agg_scr[0] * inv_col[:, 0:1]
        for h in range(1, HEADS):
            x1 = x1 + agg_scr[h] * inv_col[:, h:h + 1]
        x1 = x1 * jnp.float32(1.0 / HEADS) + bg_ref[...]
        x1 = _leaky(x1, 0.2)
        h2 = jax.lax.dot_general(x1, w2_ref[...], (((1,), (0,)), ((), ())),
                                 preferred_element_type=jnp.float32)
        dinv = jnp.float32(1.0) / jnp.sqrt(deg_scr[...])
        h2s = (h2 * dinv).astype(jnp.bfloat16)
        y = jax.lax.dot_general(b_scr[...], h2s, (((1,), (0,)), ((), ())),
                                preferred_element_type=jnp.float32)
        x2 = dinv * y + b2_ref[...]
        out_ref[0] = _leaky(x2, 0.2) * gate_ref[0]


def _cls_kernel(g_ref, wt_ref, bc_ref, out_ref):
    logits = jax.lax.dot_general(g_ref[...], wt_ref[...],
                                 (((1,), (1,)), ((), ())),
                                 preferred_element_type=jnp.float32)
    logits = logits + bc_ref[...]
    mx = jnp.max(logits, axis=1, keepdims=True)
    sh = logits - mx
    out_ref[...] = sh - jnp.log(jnp.sum(jnp.exp(sh), axis=1, keepdims=True))


@jax.jit
def kernel(features0, features1, adjs, action, index, W_gat, a_src, a_dst,
           b_gat, W_fc2, b_fc2, W_cls, b_cls):
    del features1
    idx = index.astype(jnp.int32)
    ngr = idx.shape[0]

    eye4 = jnp.eye(HEADS, dtype=jnp.float32)
    m_src = jnp.einsum('hk,hg->hkg', a_src, eye4).reshape(HEADS * HID1, HEADS)
    m_dst = jnp.einsum('hk,hg->hkg', a_dst, eye4).reshape(HEADS * HID1, HEADS)
    m_col = jnp.concatenate([m_src, m_dst], axis=1)
    m_row = m_col.T
    gate = jnp.where(action == 1, jnp.float32(1.0),
                     jnp.float32(0.0)).reshape(1,)

    grid = (ngr, NBLK + 1)
    x2_all = pl.pallas_call(
        _gcn_kernel,
        grid_spec=pltpu.PrefetchScalarGridSpec(
            num_scalar_prefetch=1,
            grid=grid,
            in_specs=[
                pl.BlockSpec(
                    (1, BLK, N),
                    lambda g, i, idx_ref: (idx_ref[g],
                                           jnp.minimum(i, NBLK - 1), 0)),
                pl.BlockSpec((1, N, F_IN),
                             lambda g, i, idx_ref: (idx_ref[g], 0, 0)),
                pl.BlockSpec((F_IN, HEADS * HID1),
                             lambda g, i, idx_ref: (0, 0)),
                pl.BlockSpec((HEADS * HID1, 2 * HEADS),
                             lambda g, i, idx_ref: (0, 0)),
                pl.BlockSpec((2 * HEADS, HEADS * HID1),
                             lambda g, i, idx_ref: (0, 0)),
                pl.BlockSpec((HID1, HID2), lambda g, i, idx_ref: (0, 0)),
                pl.BlockSpec((1, HID1), lambda g, i, idx_ref: (0, 0)),
                pl.BlockSpec((1, HID2), lambda g, i, idx_ref: (0, 0)),
                pl.BlockSpec(memory_space=pltpu.SMEM),
            ],
            out_specs=pl.BlockSpec((1, N, HID2),
                                   lambda g, i, idx_ref: (g, 0, 0)),
            scratch_shapes=[
                pltpu.VMEM((N, HEADS * HID1), jnp.bfloat16),
                pltpu.VMEM((N, 2 * HEADS), jnp.bfloat16),
                pltpu.VMEM((2 * HEADS, N), jnp.bfloat16),
                pltpu.VMEM((2 * HEADS, N), jnp.float32),
                pltpu.VMEM((HEADS, N, HID1), jnp.float32),
                pltpu.VMEM((N, 1), jnp.float32),
                pltpu.VMEM((N, N), jnp.bfloat16),
            ],
        ),
        out_shape=jax.ShapeDtypeStruct((ngr, N, HID2), jnp.float32),
        compiler_params=pltpu.CompilerParams(
            dimension_semantics=("parallel", "arbitrary")),
    )(idx, adjs, features0, W_gat, m_col, m_row, W_fc2,
      b_gat.reshape(1, HID1), b_fc2.reshape(1, HID2), gate)

    g_mat = x2_all.reshape(ngr, N * HID2)
    logp = pl.pallas_call(
        _cls_kernel,
        out_shape=jax.ShapeDtypeStruct((ngr, OUT_DIM), jnp.float32),
    )(g_mat, W_cls.T, b_cls.reshape(1, OUT_DIM))

    return (x2_all[ngr - 1], logp)


# speedup vs baseline: 1.0622x; 1.0013x over previous
import jax
import jax.numpy as jnp
from jax.experimental import pallas as pl
from jax.experimental.pallas import tpu as pltpu

N = 2048
F_IN = 90
HEADS = 4
HID1 = 64
HID2 = 128
OUT_DIM = 10
NBLK = 4
BLK = N // NBLK
LOG2E = 1.4426950408889634


def _leaky(x, slope):
    return jnp.where(x >= 0, x, slope * x)


def _gcn_kernel(idx_ref, adj_ref, x_ref, wg_ref, mcol_ref, mrow_ref, w2_ref,
                bg_ref, b2_ref, gate_ref, out_ref,
                wxp_scr, ac_scr, u_scr, den_scr, agg_scr, deg_scr,
                b_scr):
    del idx_ref
    i = pl.program_id(1)

    @pl.when(i == 0)
    def _init_graph():
        x = x_ref[0]
        wx = jax.lax.dot_general(x, wg_ref[...], (((1,), (0,)), ((), ())),
                                 preferred_element_type=jnp.float32)
        ac = jax.lax.dot_general(wx, mcol_ref[...], (((1,), (0,)), ((), ())),
                                 preferred_element_type=jnp.float32)
        ac_scr[...] = jnp.concatenate(
            [ac[:, 0:HEADS] * LOG2E, ac[:, 0:HEADS] * (0.2 * LOG2E)],
            axis=1).astype(jnp.bfloat16)
        ar = jax.lax.dot_general(mrow_ref[...], wx, (((1,), (1,)), ((), ())),
                                 preferred_element_type=jnp.float32)
        for h in range(HEADS):
            ad_r = ar[HEADS + h:HEADS + h + 1, :]
            gmax = jnp.max(ac[:, h:h + 1])
            u = _leaky(gmax + ad_r, 0.2)
            u_scr[h:h + 1, :] = ((ad_r - u) * LOG2E).astype(jnp.bfloat16)
            u_scr[HEADS + h:HEADS + h + 1, :] = (
                (ad_r * 0.2 - u) * LOG2E).astype(jnp.bfloat16)
        wxp_scr[...] = wx.astype(jnp.bfloat16)
        den_scr[...] = jnp.zeros(den_scr.shape, jnp.float32)
        agg_scr[...] = jnp.zeros(agg_scr.shape, jnp.float32)

    @pl.when(i < NBLK)
    def _attention_block():
        row0 = i * BLK
        a_blk = adj_ref[0]
        rid = row0 + jax.lax.broadcasted_iota(jnp.int32, (BLK, N), 0)
        cid = jax.lax.broadcasted_iota(jnp.int32, (BLK, N), 1)
        b_blk = jnp.where(jnp.logical_or(a_blk > 0.5, rid == cid),
                          jnp.float32(1.0), jnp.float32(0.0))
        b_scr[pl.ds(row0, BLK), :] = b_blk.astype(jnp.bfloat16)
        ones_col = jnp.full((N, 1), 1.0, jnp.float32)
        deg_scr[pl.ds(row0, BLK), :] = jax.lax.dot_general(
            b_blk, ones_col, (((1,), (0,)), ((), ())),
            preferred_element_type=jnp.float32)
        ones_row = jnp.full((1, BLK), 1.0, jnp.bfloat16)
        b16 = b_blk.astype(jnp.bfloat16)
        for h in range(HEADS):
            t1 = ac_scr[pl.ds(row0, BLK), h:h + 1] + u_scr[h:h + 1, :]
            t2 = (ac_scr[pl.ds(row0, BLK), HEADS + h:HEADS + h + 1]
                  + u_scr[HEADS + h:HEADS + h + 1, :])
            p = b16 * jnp.exp2(jnp.maximum(t1, t2))
            den_scr[h:h + 1, :] = den_scr[h:h + 1, :] + jax.lax.dot_general(
                ones_row, p, (((1,), (0,)), ((), ())),
                preferred_element_type=jnp.float32)
            wxh = wxp_scr[pl.ds(row0, BLK), pl.ds(h * HID1, HID1)]
            agg_scr[h] = agg_scr[h] + jax.lax.dot_general(
                p, wxh, (((0,), (0,)), ((), ())),
                preferred_element_type=jnp.float32)

    @pl.when(i == NBLK)
    def _finalize():
        inv = jnp.float32(1.0) / (den_scr[...] + jnp.float32(1e-16))
        eye8 = jnp.where(
            jax.lax.broadcasted_iota(jnp.int32, (8, 8), 0)
            == jax.lax.broadcasted_iota(jnp.int32, (8, 8), 1),
            jnp.float32(1.0), jnp.float32(0.0))
        inv_col = jax.lax.dot_general(inv, eye8, (((0,), (0,)), ((), ())),
                                      preferred_element_type=jnp.float32)
        x1 = agg_scr[0] * inv_col[:, 0:1]
        for h in range(1, HEADS):
            x1 = x1 + agg_scr[h] * inv_col[:, h:h + 1]
        x1 = x1 * jnp.float32(1.0 / HEADS) + bg_ref[...]
        x1 = _leaky(x1, 0.2)
        h2 = jax.lax.dot_general(x1, w2_ref[...], (((1,), (0,)), ((), ())),
                                 preferred_element_type=jnp.float32)
        dinv = jnp.float32(1.0) / jnp.sqrt(deg_scr[...])
        h2s = (h2 * dinv).astype(jnp.bfloat16)
        y = jax.lax.dot_general(b_scr[...], h2s, (((1,), (0,)), ((), ())),
                                preferred_element_type=jnp.float32)
        x2 = dinv * y + b2_ref[...]
        out_ref[0] = _leaky(x2, 0.2) * gate_ref[0]


def _cls_kernel(g_ref, wt_ref, bc_ref, out_ref):
    logits = jax.lax.dot_general(g_ref[...], wt_ref[...],
                                 (((1,), (1,)), ((), ())),
                                 preferred_element_type=jnp.float32)
    logits = logits + bc_ref[...]
    mx = jnp.max(logits, axis=1, keepdims=True)
    sh = logits - mx
    out_ref[...] = sh - jnp.log(jnp.sum(jnp.exp(sh), axis=1, keepdims=True))


@jax.jit
def kernel(features0, features1, adjs, action, index, W_gat, a_src, a_dst,
           b_gat, W_fc2, b_fc2, W_cls, b_cls):
    del features1
    idx = index.astype(jnp.int32)
    ngr = idx.shape[0]

    eye4 = jnp.eye(HEADS, dtype=jnp.float32)
    m_src = jnp.einsum('hk,hg->hkg', a_src, eye4).reshape(HEADS * HID1, HEADS)
    m_dst = jnp.einsum('hk,hg->hkg', a_dst, eye4).reshape(HEADS * HID1, HEADS)
    m_col = jnp.concatenate([m_src, m_dst], axis=1)
    m_row = m_col.T
    gate = jnp.where(action == 1, jnp.float32(1.0),
                     jnp.float32(0.0)).reshape(1,)

    grid = (ngr, NBLK + 1)
    x2_all = pl.pallas_call(
        _gcn_kernel,
        grid_spec=pltpu.PrefetchScalarGridSpec(
            num_scalar_prefetch=1,
            grid=grid,
            in_specs=[
                pl.BlockSpec(
                    (1, BLK, N),
                    lambda g, i, idx_ref: (idx_ref[g],
                                           jnp.minimum(i, NBLK - 1), 0)),
                pl.BlockSpec((1, N, F_IN),
                             lambda g, i, idx_ref: (idx_ref[g], 0, 0)),
                pl.BlockSpec((F_IN, HEADS * HID1),
                             lambda g, i, idx_ref: (0, 0)),
                pl.BlockSpec((HEADS * HID1, 2 * HEADS),
                             lambda g, i, idx_ref: (0, 0)),
                pl.BlockSpec((2 * HEADS, HEADS * HID1),
                             lambda g, i, idx_ref: (0, 0)),
                pl.BlockSpec((HID1, HID2), lambda g, i, idx_ref: (0, 0)),
                pl.BlockSpec((1, HID1), lambda g, i, idx_ref: (0, 0)),
                pl.BlockSpec((1, HID2), lambda g, i, idx_ref: (0, 0)),
                pl.BlockSpec(memory_space=pltpu.SMEM),
            ],
            out_specs=pl.BlockSpec((1, N, HID2),
                                   lambda g, i, idx_ref: (g, 0, 0)),
            scratch_shapes=[
                pltpu.VMEM((N, HEADS * HID1), jnp.bfloat16),
                pltpu.VMEM((N, 2 * HEADS), jnp.bfloat16),
                pltpu.VMEM((2 * HEADS, N), jnp.bfloat16),
                pltpu.VMEM((2 * HEADS, N), jnp.float32),
                pltpu.VMEM((HEADS, N, HID1), jnp.float32),
                pltpu.VMEM((N, 1), jnp.float32),
                pltpu.VMEM((N, N), jnp.bfloat16),
            ],
        ),
        out_shape=jax.ShapeDtypeStruct((ngr, N, HID2), jnp.float32),
        compiler_params=pltpu.CompilerParams(
            dimension_semantics=("arbitrary", "arbitrary")),
    )(idx, adjs, features0, W_gat, m_col, m_row, W_fc2,
      b_gat.reshape(1, HID1), b_fc2.reshape(1, HID2), gate)

    g_mat = x2_all.reshape(ngr, N * HID2)
    logp = pl.pallas_call(
        _cls_kernel,
        out_shape=jax.ShapeDtypeStruct((ngr, OUT_DIM), jnp.float32),
    )(g_mat, W_cls.T, b_cls.reshape(1, OUT_DIM))

    return (x2_all[ngr - 1], logp)
